# baseline (speedup 1.0000x reference)
import functools

import jax
import jax.numpy as jnp
from jax import lax
from jax.experimental import pallas as pl
from jax.experimental.pallas import tpu as pltpu
from jax.experimental.pallas import tpu_sc as plsc

N = 10000
DIM = 128
D_NODE = 64
D_GRAPH = 32
B = 64

N_CORES = 2
N_SUB = 16
N_TILES = N_CORES * N_SUB

CHUNK = 64
N_ACC = 10240
ZROWS = N_ACC // N_SUB
OUT_ROWS = N // N_SUB

NB = 4
NI = 8
KSH = 2
GROUP = 8


def _make_sc_agg(e_pad):
    edges_per_tile = e_pad // N_TILES
    chunks = edges_per_tile // CHUNK
    mesh = plsc.VectorSubcoreMesh(core_axis_name="c", subcore_axis_name="s")

    @functools.partial(
        pl.kernel,
        out_type=(
            jax.ShapeDtypeStruct((N_ACC, DIM), jnp.float32),
            jax.ShapeDtypeStruct((N_ACC, DIM), jnp.float32),
        ),
        mesh=mesh,
        scratch_types=(
            [pltpu.VMEM((CHUNK,), jnp.int32) for _ in range(NI)]
            + [pltpu.VMEM((CHUNK,), jnp.int32) for _ in range(NI)]
            + [pltpu.VMEM((CHUNK, DIM), jnp.float32) for _ in range(NB)]
            + [pltpu.VMEM_SHARED((N_ACC, DIM), jnp.float32)]
            + [pltpu.SemaphoreType.DMA for _ in range(NI)]
            + [pltpu.SemaphoreType.DMA for _ in range(NI)]
            + [pltpu.SemaphoreType.DMA for _ in range(NB)]
            + [pltpu.SemaphoreType.DMA for _ in range(NB)]
        ),
    )
    def sc_agg(h_hbm, src_hbm, dst_hbm, out0, out1, *refs):
        sidx = refs[0:NI]
        didx = refs[NI:2 * NI]
        rows = refs[2 * NI:2 * NI + NB]
        acc = refs[2 * NI + NB]
        sems = refs[2 * NI + NB + 1:]
        sem_si = sems[0:NI]
        sem_di = sems[NI:2 * NI]
        sem_g = sems[2 * NI:2 * NI + NB]
        sem_s = sems[2 * NI + NB:2 * NI + 2 * NB]

        cid = lax.axis_index("c")
        sid = lax.axis_index("s")
        wid = sid * N_CORES + cid
        ebase = pl.multiple_of(wid * edges_per_tile, 8)

        def _zrow(i, _):
            for j in range(DIM // 16):
                rows[0][i, pl.ds(j * 16, 16)] = jnp.zeros((16,), jnp.float32)
            return 0
        lax.fori_loop(0, CHUNK, _zrow, 0)
        for r in range(ZROWS // CHUNK):
            zoff = pl.multiple_of(sid * ZROWS + r * CHUNK, 8)
            pltpu.sync_copy(rows[0], acc.at[pl.ds(zoff, CHUNK)])
        plsc.subcore_barrier()

        def _idx_load(c, il):
            eoff = pl.multiple_of(ebase + c * CHUNK, 8)
            pltpu.async_copy(src_hbm.at[pl.ds(eoff, CHUNK)], sidx[il],
                             sem_si[il])
            pltpu.async_copy(dst_hbm.at[pl.ds(eoff, CHUNK)], didx[il],
                             sem_di[il])

        for c in range(NB):
            _idx_load(c, c)

        total = -(-(chunks + NB) // GROUP) * GROUP

        def _group(g, _):
            for u in range(GROUP):
                i = g * GROUP + u
                b = u % NB
                il = u % NI
                bk = (u - KSH) % NB
                ilk = (u - KSH) % NI
                iln = (u + NB) % NI

                @pl.when(jnp.logical_and(i >= NB, i < chunks + NB))
                def _():
                    pltpu.make_async_copy(
                        rows[b], acc.at[didx[iln]], sem_s[b]).wait()

                @pl.when(i + NB < chunks)
                def _():
                    _idx_load(i + NB, iln)

                @pl.when(i < chunks)
                def _():
                    pltpu.make_async_copy(
                        src_hbm.at[pl.ds(0, CHUNK)], sidx[il],
                        sem_si[il]).wait()
                    pltpu.make_async_copy(
                        dst_hbm.at[pl.ds(0, CHUNK)], didx[il],
                        sem_di[il]).wait()
                    pltpu.async_copy(h_hbm.at[sidx[il]], rows[b], sem_g[b])

                @pl.when(jnp.logical_and(i >= KSH, i < chunks + KSH))
                def _():
                    pltpu.make_async_copy(
                        h_hbm.at[sidx[ilk]], rows[bk], sem_g[bk]).wait()
                    pltpu.async_copy(rows[bk], acc.at[didx[ilk]], sem_s[bk],
                                     add=True)
            return 0
        lax.fori_loop(0, total // GROUP, _group, 0)
        plsc.subcore_barrier()

        row0 = pl.multiple_of(sid * ZROWS, 8)
        @pl.when(cid == 0)
        def _():
            pltpu.sync_copy(acc.at[pl.ds(row0, ZROWS)],
                            out0.at[pl.ds(row0, ZROWS)])
        @pl.when(cid == 1)
        def _():
            pltpu.sync_copy(acc.at[pl.ds(row0, ZROWS)],
                            out1.at[pl.ds(row0, ZROWS)])

    return sc_agg


def _tc_layer(h, agg0, agg1, wa, ba, wb, bb, scale, shift):
    rows = 2000
    dh = wa.shape[1]
    dout = wb.shape[1]

    def body(h_ref, a0_ref, a1_ref, wa_ref, ba_ref, wb_ref, bb_ref,
             sc_ref, sh_ref, o_ref):
        m = h_ref[...] + a0_ref[...] + a1_ref[...]
        u = jnp.dot(m, wa_ref[...], preferred_element_type=jnp.float32)
        u = u + ba_ref[...]
        u = jnp.where(u > 0, u, jnp.exp(u) - 1.0)
        v = jnp.dot(u, wb_ref[...], preferred_element_type=jnp.float32)
        v = v + bb_ref[...]
        v = jnp.where(v > 0, v, jnp.exp(v) - 1.0)
        o_ref[...] = v * sc_ref[...] + sh_ref[...]

    full = lambda shape: pl.BlockSpec(shape, lambda i: (0,) * len(shape))
    return pl.pallas_call(
        body,
        grid=(N // rows,),
        in_specs=[
            pl.BlockSpec((rows, DIM), lambda i: (i, 0)),
            pl.BlockSpec((rows, DIM), lambda i: (i, 0)),
            pl.BlockSpec((rows, DIM), lambda i: (i, 0)),
            full((DIM, dh)),
            full((1, dh)),
            full((dh, dout)),
            full((1, dout)),
            full((1, dout)),
            full((1, dout)),
        ],
        out_specs=pl.BlockSpec((rows, dout), lambda i: (i, 0)),
        out_shape=jax.ShapeDtypeStruct((N, dout), jnp.float32),
    )(h, agg0, agg1, wa, ba.reshape(1, -1), wb, bb.reshape(1, -1),
      scale.reshape(1, -1), shift.reshape(1, -1))


def _tc_final(h5p, batch2d, wf1, bf1, wf2, bf2):
    npad = h5p.shape[0]

    def body(h_ref, b_ref, w1_ref, b1_ref, w2_ref, b2_ref, o_ref):
        brow = b_ref[0:1, :]
        ids = lax.broadcasted_iota(jnp.int32, (B, npad), 0)
        mask = (ids == brow).astype(jnp.float32)
        pooled = jnp.dot(mask, h_ref[...],
                         preferred_element_type=jnp.float32)
        z = jnp.dot(pooled, w1_ref[...],
                    preferred_element_type=jnp.float32) + b1_ref[...]
        z = jnp.where(z > 0, z, jnp.exp(z) - 1.0)
        z2 = jnp.dot(z, w2_ref[...],
                     preferred_element_type=jnp.float32) + b2_ref[...]
        gg = D_GRAPH * D_GRAPH
        bi = lax.broadcasted_iota(jnp.int32, (D_GRAPH, gg), 0)
        bk = lax.broadcasted_iota(jnp.int32, (D_GRAPH, gg), 1)
        r1 = (bk // D_GRAPH == bi).astype(jnp.float32)
        r2 = (bk % D_GRAPH == bi).astype(jnp.float32)
        a = (jnp.dot(z2, r1, preferred_element_type=jnp.float32)
             * jnp.dot(z2, r2, preferred_element_type=jnp.float32))
        o_ref[...] = a[:, :, None] * pooled[:, None, :]

    full = lambda shape: pl.BlockSpec(shape, lambda: (0,) * len(shape))
    return pl.pallas_call(
        body,
        in_specs=[
            full((npad, D_NODE)),
            full((8, npad)),
            full((D_NODE, D_NODE)),
            full((1, D_NODE)),
            full((D_NODE, D_GRAPH)),
            full((1, D_GRAPH)),
        ],
        out_specs=full((B, D_GRAPH * D_GRAPH, D_NODE)),
        out_shape=jax.ShapeDtypeStruct((B, D_GRAPH * D_GRAPH, D_NODE),
                                       jnp.float32),
    )(h5p, batch2d, wf1, bf1.reshape(1, -1), wf2, bf2.reshape(1, -1))


def kernel(x, edge_index, batch, params, stats):
    p, st = params, stats
    e = edge_index.shape[1]
    e_pad = -(-e // (N_TILES * CHUNK * 8)) * (N_TILES * CHUNK * 8)
    order = jnp.argsort(edge_index[0])
    src = jnp.concatenate(
        [edge_index[0][order], jnp.zeros((e_pad - e,), jnp.int32)])
    dst = jnp.concatenate(
        [edge_index[1][order], jnp.full((e_pad - e,), N_ACC - 1, jnp.int32)])

    sc_agg = _make_sc_agg(e_pad)

    h = x
    for i in range(1, 6):
        g, be = p['g%d' % i], p['be%d' % i]
        rm, rv = st['rm%d' % i], st['rv%d' % i]
        scale = g * lax.rsqrt(rv + 1e-5)
        shift = be - rm * scale
        agg0, agg1 = sc_agg(h, src, dst)
        h = _tc_layer(h, agg0, agg1, p['w%da' % i], p['b%da' % i],
                      p['w%db' % i], p['b%db' % i], scale, shift)

    npad = N_ACC
    h5p = jnp.concatenate(
        [h, jnp.zeros((npad - N, D_NODE), jnp.float32)])
    bpad = jnp.concatenate(
        [batch, jnp.full((npad - N,), B, jnp.int32)])
    batch2d = jnp.broadcast_to(bpad[None, :], (8, npad))

    out = _tc_final(h5p, batch2d, p['wf1'], p['bf1'], p['wf2'], p['bf2'])
    return out.reshape(B, D_GRAPH, D_GRAPH, D_NODE)


# speedup vs baseline: 1.1853x; 1.1853x over previous
import functools

import jax
import jax.numpy as jnp
from jax import lax
from jax.experimental import pallas as pl
from jax.experimental.pallas import tpu as pltpu
from jax.experimental.pallas import tpu_sc as plsc

N = 10000
DIM = 128
D_NODE = 64
D_GRAPH = 32
B = 64

N_CORES = 2
N_SUB = 16
N_TILES = N_CORES * N_SUB

CHUNK = 128
N_ACC = 10240
ZROWS = N_ACC // N_SUB
OUT_ROWS = N // N_SUB

NB = 2
NI = 4
KSH = 1
GROUP = 4


def _make_sc_agg(e_pad):
    edges_per_tile = e_pad // N_TILES
    chunks = edges_per_tile // CHUNK
    mesh = plsc.VectorSubcoreMesh(core_axis_name="c", subcore_axis_name="s")

    @functools.partial(
        pl.kernel,
        out_type=(
            jax.ShapeDtypeStruct((N_ACC, DIM), jnp.float32),
            jax.ShapeDtypeStruct((N_ACC, DIM), jnp.float32),
        ),
        mesh=mesh,
        scratch_types=(
            [pltpu.VMEM((CHUNK,), jnp.int32) for _ in range(NI)]
            + [pltpu.VMEM((CHUNK,), jnp.int32) for _ in range(NI)]
            + [pltpu.VMEM((CHUNK, DIM), jnp.float32) for _ in range(NB)]
            + [pltpu.VMEM_SHARED((N_ACC, DIM), jnp.float32)]
            + [pltpu.SemaphoreType.DMA for _ in range(NI)]
            + [pltpu.SemaphoreType.DMA for _ in range(NI)]
            + [pltpu.SemaphoreType.DMA for _ in range(NB)]
            + [pltpu.SemaphoreType.DMA for _ in range(NB)]
        ),
    )
    def sc_agg(h_hbm, src_hbm, dst_hbm, out0, out1, *refs):
        sidx = refs[0:NI]
        didx = refs[NI:2 * NI]
        rows = refs[2 * NI:2 * NI + NB]
        acc = refs[2 * NI + NB]
        sems = refs[2 * NI + NB + 1:]
        sem_si = sems[0:NI]
        sem_di = sems[NI:2 * NI]
        sem_g = sems[2 * NI:2 * NI + NB]
        sem_s = sems[2 * NI + NB:2 * NI + 2 * NB]

        cid = lax.axis_index("c")
        sid = lax.axis_index("s")
        wid = sid * N_CORES + cid
        ebase = pl.multiple_of(wid * edges_per_tile, 8)

        def _zrow(i, _):
            for j in range(DIM // 16):
                rows[0][i, pl.ds(j * 16, 16)] = jnp.zeros((16,), jnp.float32)
            return 0
        lax.fori_loop(0, CHUNK, _zrow, 0)
        for r in range(ZROWS // CHUNK):
            zoff = pl.multiple_of(sid * ZROWS + r * CHUNK, 8)
            pltpu.sync_copy(rows[0], acc.at[pl.ds(zoff, CHUNK)])
        plsc.subcore_barrier()

        def _idx_load(c, il):
            eoff = pl.multiple_of(ebase + c * CHUNK, 8)
            pltpu.async_copy(src_hbm.at[pl.ds(eoff, CHUNK)], sidx[il],
                             sem_si[il])
            pltpu.async_copy(dst_hbm.at[pl.ds(eoff, CHUNK)], didx[il],
                             sem_di[il])

        for c in range(NB):
            _idx_load(c, c)

        total = -(-(chunks + NB) // GROUP) * GROUP

        def _group(g, _):
            for u in range(GROUP):
                i = g * GROUP + u
                b = u % NB
                il = u % NI
                bk = (u - KSH) % NB
                ilk = (u - KSH) % NI
                iln = (u + NB) % NI

                @pl.when(jnp.logical_and(i >= NB, i < chunks + NB))
                def _():
                    pltpu.make_async_copy(
                        rows[b], acc.at[didx[iln]], sem_s[b]).wait()

                @pl.when(i + NB < chunks)
                def _():
                    _idx_load(i + NB, iln)

                @pl.when(i < chunks)
                def _():
                    pltpu.make_async_copy(
                        src_hbm.at[pl.ds(0, CHUNK)], sidx[il],
                        sem_si[il]).wait()
                    pltpu.make_async_copy(
                        dst_hbm.at[pl.ds(0, CHUNK)], didx[il],
                        sem_di[il]).wait()
                    pltpu.async_copy(h_hbm.at[sidx[il]], rows[b], sem_g[b])

                @pl.when(jnp.logical_and(i >= KSH, i < chunks + KSH))
                def _():
                    pltpu.make_async_copy(
                        h_hbm.at[sidx[ilk]], rows[bk], sem_g[bk]).wait()
                    pltpu.async_copy(rows[bk], acc.at[didx[ilk]], sem_s[bk],
                                     add=True)
            return 0
        lax.fori_loop(0, total // GROUP, _group, 0)
        plsc.subcore_barrier()

        row0 = pl.multiple_of(sid * ZROWS, 8)
        @pl.when(cid == 0)
        def _():
            pltpu.sync_copy(acc.at[pl.ds(row0, ZROWS)],
                            out0.at[pl.ds(row0, ZROWS)])
        @pl.when(cid == 1)
        def _():
            pltpu.sync_copy(acc.at[pl.ds(row0, ZROWS)],
                            out1.at[pl.ds(row0, ZROWS)])

    return sc_agg


def _tc_layer(h, agg0, agg1, wa, ba, wb, bb, scale, shift):
    rows = 2000
    dh = wa.shape[1]
    dout = wb.shape[1]

    def body(h_ref, a0_ref, a1_ref, wa_ref, ba_ref, wb_ref, bb_ref,
             sc_ref, sh_ref, o_ref):
        m = h_ref[...] + a0_ref[...] + a1_ref[...]
        u = jnp.dot(m, wa_ref[...], preferred_element_type=jnp.float32)
        u = u + ba_ref[...]
        u = jnp.where(u > 0, u, jnp.exp(u) - 1.0)
        v = jnp.dot(u, wb_ref[...], preferred_element_type=jnp.float32)
        v = v + bb_ref[...]
        v = jnp.where(v > 0, v, jnp.exp(v) - 1.0)
        o_ref[...] = v * sc_ref[...] + sh_ref[...]

    full = lambda shape: pl.BlockSpec(shape, lambda i: (0,) * len(shape))
    return pl.pallas_call(
        body,
        grid=(N // rows,),
        in_specs=[
            pl.BlockSpec((rows, DIM), lambda i: (i, 0)),
            pl.BlockSpec((rows, DIM), lambda i: (i, 0)),
            pl.BlockSpec((rows, DIM), lambda i: (i, 0)),
            full((DIM, dh)),
            full((1, dh)),
            full((dh, dout)),
            full((1, dout)),
            full((1, dout)),
            full((1, dout)),
        ],
        out_specs=pl.BlockSpec((rows, dout), lambda i: (i, 0)),
        out_shape=jax.ShapeDtypeStruct((N, dout), jnp.float32),
    )(h, agg0, agg1, wa, ba.reshape(1, -1), wb, bb.reshape(1, -1),
      scale.reshape(1, -1), shift.reshape(1, -1))


def _tc_final(h5p, batch2d, wf1, bf1, wf2, bf2):
    npad = h5p.shape[0]

    def body(h_ref, b_ref, w1_ref, b1_ref, w2_ref, b2_ref, o_ref):
        brow = b_ref[0:1, :]
        ids = lax.broadcasted_iota(jnp.int32, (B, npad), 0)
        mask = (ids == brow).astype(jnp.float32)
        pooled = jnp.dot(mask, h_ref[...],
                         preferred_element_type=jnp.float32)
        z = jnp.dot(pooled, w1_ref[...],
                    preferred_element_type=jnp.float32) + b1_ref[...]
        z = jnp.where(z > 0, z, jnp.exp(z) - 1.0)
        z2 = jnp.dot(z, w2_ref[...],
                     preferred_element_type=jnp.float32) + b2_ref[...]
        gg = D_GRAPH * D_GRAPH
        bi = lax.broadcasted_iota(jnp.int32, (D_GRAPH, gg), 0)
        bk = lax.broadcasted_iota(jnp.int32, (D_GRAPH, gg), 1)
        r1 = (bk // D_GRAPH == bi).astype(jnp.float32)
        r2 = (bk % D_GRAPH == bi).astype(jnp.float32)
        a = (jnp.dot(z2, r1, preferred_element_type=jnp.float32)
             * jnp.dot(z2, r2, preferred_element_type=jnp.float32))
        o_ref[...] = a[:, :, None] * pooled[:, None, :]

    full = lambda shape: pl.BlockSpec(shape, lambda: (0,) * len(shape))
    return pl.pallas_call(
        body,
        in_specs=[
            full((npad, D_NODE)),
            full((8, npad)),
            full((D_NODE, D_NODE)),
            full((1, D_NODE)),
            full((D_NODE, D_GRAPH)),
            full((1, D_GRAPH)),
        ],
        out_specs=full((B, D_GRAPH * D_GRAPH, D_NODE)),
        out_shape=jax.ShapeDtypeStruct((B, D_GRAPH * D_GRAPH, D_NODE),
                                       jnp.float32),
    )(h5p, batch2d, wf1, bf1.reshape(1, -1), wf2, bf2.reshape(1, -1))


def kernel(x, edge_index, batch, params, stats):
    p, st = params, stats
    e = edge_index.shape[1]
    e_pad = -(-e // (N_TILES * CHUNK * 8)) * (N_TILES * CHUNK * 8)
    src = jnp.concatenate(
        [edge_index[0], jnp.zeros((e_pad - e,), jnp.int32)])
    dst = jnp.concatenate(
        [edge_index[1], jnp.full((e_pad - e,), N_ACC - 1, jnp.int32)])

    sc_agg = _make_sc_agg(e_pad)

    h = x
    for i in range(1, 6):
        g, be = p['g%d' % i], p['be%d' % i]
        rm, rv = st['rm%d' % i], st['rv%d' % i]
        scale = g * lax.rsqrt(rv + 1e-5)
        shift = be - rm * scale
        agg0, agg1 = sc_agg(h, src, dst)
        h = _tc_layer(h, agg0, agg1, p['w%da' % i], p['b%da' % i],
                      p['w%db' % i], p['b%db' % i], scale, shift)

    npad = N_ACC
    h5p = jnp.concatenate(
        [h, jnp.zeros((npad - N, D_NODE), jnp.float32)])
    bpad = jnp.concatenate(
        [batch, jnp.full((npad - N,), B, jnp.int32)])
    batch2d = jnp.broadcast_to(bpad[None, :], (8, npad))

    out = _tc_final(h5p, batch2d, p['wf1'], p['bf1'], p['wf2'], p['bf2'])
    return out.reshape(B, D_GRAPH, D_GRAPH, D_NODE)


# speedup vs baseline: 1.1860x; 1.0006x over previous
import functools

import jax
import jax.numpy as jnp
from jax import lax
from jax.experimental import pallas as pl
from jax.experimental.pallas import tpu as pltpu
from jax.experimental.pallas import tpu_sc as plsc

N = 10000
DIM = 128
D_NODE = 64
D_GRAPH = 32
B = 64

N_CORES = 2
N_SUB = 16
N_TILES = N_CORES * N_SUB

CHUNK = 64
N_ACC = 10240
ZROWS = N_ACC // N_SUB
OUT_ROWS = N // N_SUB

NB = 4
NI = 8
KSH = 2
GROUP = 8


def _make_sc_agg(e_pad):
    edges_per_tile = e_pad // N_TILES
    chunks = edges_per_tile // CHUNK
    mesh = plsc.VectorSubcoreMesh(core_axis_name="c", subcore_axis_name="s")

    @functools.partial(
        pl.kernel,
        out_type=(
            jax.ShapeDtypeStruct((N_ACC, DIM), jnp.float32),
            jax.ShapeDtypeStruct((N_ACC, DIM), jnp.float32),
        ),
        mesh=mesh,
        scratch_types=(
            [pltpu.VMEM((CHUNK,), jnp.int32) for _ in range(NI)]
            + [pltpu.VMEM((CHUNK,), jnp.int32) for _ in range(NI)]
            + [pltpu.VMEM((CHUNK, DIM), jnp.float32) for _ in range(NB)]
            + [pltpu.VMEM_SHARED((N_ACC, DIM), jnp.float32)]
            + [pltpu.SemaphoreType.DMA for _ in range(NI)]
            + [pltpu.SemaphoreType.DMA for _ in range(NI)]
            + [pltpu.SemaphoreType.DMA for _ in range(NB)]
            + [pltpu.SemaphoreType.DMA for _ in range(NB)]
        ),
    )
    def sc_agg(h_hbm, src_hbm, dst_hbm, out0, out1, *refs):
        sidx = refs[0:NI]
        didx = refs[NI:2 * NI]
        rows = refs[2 * NI:2 * NI + NB]
        acc = refs[2 * NI + NB]
        sems = refs[2 * NI + NB + 1:]
        sem_si = sems[0:NI]
        sem_di = sems[NI:2 * NI]
        sem_g = sems[2 * NI:2 * NI + NB]
        sem_s = sems[2 * NI + NB:2 * NI + 2 * NB]

        cid = lax.axis_index("c")
        sid = lax.axis_index("s")
        wid = sid * N_CORES + cid
        ebase = pl.multiple_of(wid * edges_per_tile, 8)

        def _zrow(i, _):
            for j in range(DIM // 16):
                rows[0][i, pl.ds(j * 16, 16)] = jnp.zeros((16,), jnp.float32)
            return 0
        lax.fori_loop(0, CHUNK, _zrow, 0)
        for r in range(ZROWS // CHUNK):
            zoff = pl.multiple_of(sid * ZROWS + r * CHUNK, 8)
            pltpu.sync_copy(rows[0], acc.at[pl.ds(zoff, CHUNK)])
        plsc.subcore_barrier()

        def _idx_load(c, il):
            eoff = pl.multiple_of(ebase + c * CHUNK, 8)
            pltpu.async_copy(src_hbm.at[pl.ds(eoff, CHUNK)], sidx[il],
                             sem_si[il])
            pltpu.async_copy(dst_hbm.at[pl.ds(eoff, CHUNK)], didx[il],
                             sem_di[il])

        for c in range(NB):
            _idx_load(c, c)

        total = -(-(chunks + NB) // GROUP) * GROUP

        def _group(g, _):
            for u in range(GROUP):
                i = g * GROUP + u
                b = u % NB
                il = u % NI
                bk = (u - KSH) % NB
                ilk = (u - KSH) % NI
                iln = (u + NB) % NI

                @pl.when(jnp.logical_and(i >= NB, i < chunks + NB))
                def _():
                    pltpu.make_async_copy(
                        rows[b], acc.at[didx[iln]], sem_s[b]).wait()

                @pl.when(i + NB < chunks)
                def _():
                    _idx_load(i + NB, iln)

                @pl.when(i < chunks)
                def _():
                    pltpu.make_async_copy(
                        src_hbm.at[pl.ds(0, CHUNK)], sidx[il],
                        sem_si[il]).wait()
                    pltpu.make_async_copy(
                        dst_hbm.at[pl.ds(0, CHUNK)], didx[il],
                        sem_di[il]).wait()
                    pltpu.async_copy(h_hbm.at[sidx[il]], rows[b], sem_g[b])

                @pl.when(jnp.logical_and(i >= KSH, i < chunks + KSH))
                def _():
                    pltpu.make_async_copy(
                        h_hbm.at[sidx[ilk]], rows[bk], sem_g[bk]).wait()
                    pltpu.async_copy(rows[bk], acc.at[didx[ilk]], sem_s[bk],
                                     add=True)
            return 0
        lax.fori_loop(0, total // GROUP, _group, 0)
        plsc.subcore_barrier()

        row0 = pl.multiple_of(sid * ZROWS, 8)
        @pl.when(cid == 0)
        def _():
            pltpu.sync_copy(acc.at[pl.ds(row0, ZROWS)],
                            out0.at[pl.ds(row0, ZROWS)])
        @pl.when(cid == 1)
        def _():
            pltpu.sync_copy(acc.at[pl.ds(row0, ZROWS)],
                            out1.at[pl.ds(row0, ZROWS)])

    return sc_agg


def _tc_layer(h, agg0, agg1, wa, ba, wb, bb, scale, shift):
    rows = 2000
    dh = wa.shape[1]
    dout = wb.shape[1]

    def body(h_ref, a0_ref, a1_ref, wa_ref, ba_ref, wb_ref, bb_ref,
             sc_ref, sh_ref, o_ref):
        m = h_ref[...] + a0_ref[...] + a1_ref[...]
        u = jnp.dot(m, wa_ref[...], preferred_element_type=jnp.float32)
        u = u + ba_ref[...]
        u = jnp.where(u > 0, u, jnp.exp(u) - 1.0)
        v = jnp.dot(u, wb_ref[...], preferred_element_type=jnp.float32)
        v = v + bb_ref[...]
        v = jnp.where(v > 0, v, jnp.exp(v) - 1.0)
        o_ref[...] = v * sc_ref[...] + sh_ref[...]

    full = lambda shape: pl.BlockSpec(shape, lambda i: (0,) * len(shape))
    return pl.pallas_call(
        body,
        grid=(N // rows,),
        in_specs=[
            pl.BlockSpec((rows, DIM), lambda i: (i, 0)),
            pl.BlockSpec((rows, DIM), lambda i: (i, 0)),
            pl.BlockSpec((rows, DIM), lambda i: (i, 0)),
            full((DIM, dh)),
            full((1, dh)),
            full((dh, dout)),
            full((1, dout)),
            full((1, dout)),
            full((1, dout)),
        ],
        out_specs=pl.BlockSpec((rows, dout), lambda i: (i, 0)),
        out_shape=jax.ShapeDtypeStruct((N, dout), jnp.float32),
    )(h, agg0, agg1, wa, ba.reshape(1, -1), wb, bb.reshape(1, -1),
      scale.reshape(1, -1), shift.reshape(1, -1))


def _tc_final(h5p, batch2d, wf1, bf1, wf2, bf2):
    npad = h5p.shape[0]

    def body(h_ref, b_ref, w1_ref, b1_ref, w2_ref, b2_ref, o_ref):
        brow = b_ref[0:1, :]
        ids = lax.broadcasted_iota(jnp.int32, (B, npad), 0)
        mask = (ids == brow).astype(jnp.float32)
        pooled = jnp.dot(mask, h_ref[...],
                         preferred_element_type=jnp.float32)
        z = jnp.dot(pooled, w1_ref[...],
                    preferred_element_type=jnp.float32) + b1_ref[...]
        z = jnp.where(z > 0, z, jnp.exp(z) - 1.0)
        z2 = jnp.dot(z, w2_ref[...],
                     preferred_element_type=jnp.float32) + b2_ref[...]
        gg = D_GRAPH * D_GRAPH
        bi = lax.broadcasted_iota(jnp.int32, (D_GRAPH, gg), 0)
        bk = lax.broadcasted_iota(jnp.int32, (D_GRAPH, gg), 1)
        r1 = (bk // D_GRAPH == bi).astype(jnp.float32)
        r2 = (bk % D_GRAPH == bi).astype(jnp.float32)
        a = (jnp.dot(z2, r1, preferred_element_type=jnp.float32)
             * jnp.dot(z2, r2, preferred_element_type=jnp.float32))
        o_ref[...] = a[:, :, None] * pooled[:, None, :]

    full = lambda shape: pl.BlockSpec(shape, lambda: (0,) * len(shape))
    return pl.pallas_call(
        body,
        in_specs=[
            full((npad, D_NODE)),
            full((8, npad)),
            full((D_NODE, D_NODE)),
            full((1, D_NODE)),
            full((D_NODE, D_GRAPH)),
            full((1, D_GRAPH)),
        ],
        out_specs=full((B, D_GRAPH * D_GRAPH, D_NODE)),
        out_shape=jax.ShapeDtypeStruct((B, D_GRAPH * D_GRAPH, D_NODE),
                                       jnp.float32),
    )(h5p, batch2d, wf1, bf1.reshape(1, -1), wf2, bf2.reshape(1, -1))


def kernel(x, edge_index, batch, params, stats):
    p, st = params, stats
    e = edge_index.shape[1]
    e_pad = -(-e // (N_TILES * CHUNK * 8)) * (N_TILES * CHUNK * 8)
    src = jnp.concatenate(
        [edge_index[0], jnp.zeros((e_pad - e,), jnp.int32)])
    dst = jnp.concatenate(
        [edge_index[1], jnp.full((e_pad - e,), N_ACC - 1, jnp.int32)])

    sc_agg = _make_sc_agg(e_pad)

    h = x
    for i in range(1, 6):
        g, be = p['g%d' % i], p['be%d' % i]
        rm, rv = st['rm%d' % i], st['rv%d' % i]
        scale = g * lax.rsqrt(rv + 1e-5)
        shift = be - rm * scale
        agg0, agg1 = sc_agg(h, src, dst)
        h = _tc_layer(h, agg0, agg1, p['w%da' % i], p['b%da' % i],
                      p['w%db' % i], p['b%db' % i], scale, shift)

    npad = N_ACC
    h5p = jnp.concatenate(
        [h, jnp.zeros((npad - N, D_NODE), jnp.float32)])
    bpad = jnp.concatenate(
        [batch, jnp.full((npad - N,), B, jnp.int32)])
    batch2d = jnp.broadcast_to(bpad[None, :], (8, npad))

    out = _tc_final(h5p, batch2d, p['wf1'], p['bf1'], p['wf2'], p['bf2'])
    return out.reshape(B, D_GRAPH, D_GRAPH, D_NODE)
